# baseline (speedup 1.0000x reference)
import functools

import jax
import jax.numpy as jnp
from jax import lax
from jax.experimental import pallas as pl
from jax.experimental.pallas import tpu as pltpu
from jax.experimental.pallas import tpu_sc as plsc

NC = 2
NS = 16
NW = NC * NS
K = 128
GRP = 8
BN_EPS = 1e-5
BLK = 1000


def _sc_mesh():
    return plsc.VectorSubcoreMesh(
        core_axis_name="c", subcore_axis_name="s", num_cores=NC, num_subcores=NS
    )


def _make_agg_kernel(ch_per_w, n_acc, d):
    rpt = n_acc // NS

    assert ch_per_w % GRP == 0

    @functools.partial(
        pl.kernel,
        out_type=jax.ShapeDtypeStruct((NC, n_acc, d), jnp.float32),
        mesh=_sc_mesh(),
        scratch_types=[
            pltpu.VMEM((GRP, 2, K), jnp.int32),
            pltpu.VMEM((K, d), jnp.float32),
            pltpu.VMEM_SHARED((n_acc, d), jnp.float32),
            pltpu.SemaphoreType.DMA,
        ],
    )
    def agg_kernel(ap_hbm, eidx_hbm, zeros_hbm, out_hbm, idx, rows, acc, sem):
        c = lax.axis_index("c")
        s = lax.axis_index("s")
        w = s * NC + c
        pltpu.sync_copy(zeros_hbm, acc.at[pl.ds(s * rpt, rpt)])
        plsc.subcore_barrier()

        def body(g, carry):
            pltpu.sync_copy(eidx_hbm.at[w, pl.ds(g * GRP, GRP)], idx)
            for b in range(GRP):
                pltpu.async_copy(ap_hbm.at[idx.at[b, 0]], rows, sem).wait()
                pltpu.sync_copy(rows, acc.at[idx.at[b, 1]], add=True)
            return carry

        lax.fori_loop(0, ch_per_w // GRP, body, 0)
        plsc.subcore_barrier()
        pltpu.sync_copy(
            acc.at[pl.ds(s * rpt, rpt)], out_hbm.at[c, pl.ds(s * rpt, rpt)]
        )

    return agg_kernel


def _make_deg_kernel(ch_per_w, n_acc, d):
    rpt = n_acc // NS

    assert ch_per_w % GRP == 0

    @functools.partial(
        pl.kernel,
        out_type=jax.ShapeDtypeStruct((NC, n_acc, d), jnp.float32),
        mesh=_sc_mesh(),
        scratch_types=[
            pltpu.VMEM((GRP, 2, K), jnp.int32),
            pltpu.VMEM((K, d), jnp.float32),
            pltpu.VMEM_SHARED((n_acc, d), jnp.float32),
        ],
    )
    def deg_kernel(eidx_hbm, ones_hbm, zeros_hbm, out_hbm, idx, ones_v, acc):
        c = lax.axis_index("c")
        s = lax.axis_index("s")
        w = s * NC + c
        pltpu.sync_copy(ones_hbm, ones_v)
        pltpu.sync_copy(zeros_hbm, acc.at[pl.ds(s * rpt, rpt)])
        plsc.subcore_barrier()

        def body(g, carry):
            pltpu.sync_copy(eidx_hbm.at[w, pl.ds(g * GRP, GRP)], idx)
            for b in range(GRP):
                pltpu.sync_copy(ones_v, acc.at[idx.at[b, 1]], add=True)
            return carry

        lax.fori_loop(0, ch_per_w // GRP, body, 0)
        plsc.subcore_barrier()
        pltpu.sync_copy(
            acc.at[pl.ds(s * rpt, rpt)], out_hbm.at[c, pl.ds(s * rpt, rpt)]
        )

    return deg_kernel


def _entry_body(dega_ref, x_ref, wg_ref, dis_ref, ap_ref):
    dw = dega_ref[...]
    deg = dw[0, :, 0:1] + dw[1, :, 0:1] + 1.0
    dis = lax.rsqrt(deg)
    dis_ref[...] = dis
    ap_ref[...] = (
        jnp.dot(x_ref[...], wg_ref[...], preferred_element_type=jnp.float32) * dis
    )


def _combine_body(agg_ref, ap_ref, dis_ref, h_ref, wl_ref, wgn_ref,
                  bg_ref, bl_ref, mean_ref, var_ref, gamma_ref, beta_ref,
                  hn_ref, apn_ref):
    ag = agg_ref[...]
    dis = dis_ref[...]
    pre = (ag[0] + ag[1] + ap_ref[...]) * dis + bg_ref[...]
    pre += jnp.dot(h_ref[...], wl_ref[...], preferred_element_type=jnp.float32)
    pre += bl_ref[...]
    scale = gamma_ref[...] * lax.rsqrt(var_ref[...] + BN_EPS)
    hn = jnp.maximum((pre - mean_ref[...]) * scale + beta_ref[...], 0.0)
    hn_ref[...] = hn
    apn_ref[...] = (
        jnp.dot(hn, wgn_ref[...], preferred_element_type=jnp.float32) * dis
    )


def _final_body(agg_ref, ap_ref, dis_ref, h_ref, wl_ref, wp_ref,
                bg_ref, bl_ref, mean_ref, var_ref, gamma_ref, beta_ref,
                bp_ref, y_ref):
    ag = agg_ref[...]
    dis = dis_ref[...]
    pre = (ag[0] + ag[1] + ap_ref[...]) * dis + bg_ref[...]
    pre += jnp.dot(h_ref[...], wl_ref[...], preferred_element_type=jnp.float32)
    pre += bl_ref[...]
    scale = gamma_ref[...] * lax.rsqrt(var_ref[...] + BN_EPS)
    hn = jnp.maximum((pre - mean_ref[...]) * scale + beta_ref[...], 0.0)
    y_ref[...] = (
        jnp.dot(hn, wp_ref[...], preferred_element_type=jnp.float32) + bp_ref[...]
    )


def _row_spec(b, d):
    return pl.BlockSpec((b, d), lambda i: (i, 0))


def _full_spec(shape):
    ndim = len(shape)
    return pl.BlockSpec(shape, lambda i: (0,) * ndim)


def kernel(x, edge_index, Wg, bg, Wl, bl, gamma, beta, running_mean,
           running_var, Wp, bp):
    n, d = x.shape
    e = edge_index.shape[1]
    d_out = Wp.shape[1]
    n_layers = Wg.shape[0]

    ch_per_w = -(-e // (NW * K))
    ch_per_w = -(-ch_per_w // GRP) * GRP
    e_pad = NW * ch_per_w * K
    n_acc = -(-(n + 1) // (NS * 8)) * (NS * 8)
    rpt = n_acc // NS

    src = edge_index[0].astype(jnp.int32)
    dst = edge_index[1].astype(jnp.int32)
    pad = e_pad - e
    esrc = jnp.concatenate([src, jnp.zeros((pad,), jnp.int32)]).reshape(NW, ch_per_w, K)
    edst = jnp.concatenate([dst, jnp.full((pad,), n, jnp.int32)]).reshape(NW, ch_per_w, K)
    eidx = jnp.stack([esrc, edst], axis=2)

    zeros_agg = jnp.zeros((rpt, d), jnp.float32)
    ones_kd = jnp.ones((K, d), jnp.float32)

    agg_kernel = _make_agg_kernel(ch_per_w, n_acc, d)

    dega = _make_deg_kernel(ch_per_w, n_acc, d)(eidx, ones_kd, zeros_agg)

    grid = (n // BLK,)
    dis, ap = pl.pallas_call(
        _entry_body,
        grid=grid,
        in_specs=[
            pl.BlockSpec((NC, BLK, d), lambda i: (0, i, 0)),
            _row_spec(BLK, d),
            _full_spec((d, d)),
        ],
        out_specs=[_row_spec(BLK, 1), _row_spec(BLK, d)],
        out_shape=[
            jax.ShapeDtypeStruct((n, 1), jnp.float32),
            jax.ShapeDtypeStruct((n, d), jnp.float32),
        ],
    )(dega, x, Wg[0])

    vec = lambda a: a.reshape(1, -1)
    h = x
    for i in range(n_layers):
        agg = agg_kernel(ap, eidx, zeros_agg)
        last = i == n_layers - 1
        vspecs = [_full_spec((1, d))] * 6
        common_in = [
            pl.BlockSpec((NC, BLK, d), lambda i_: (0, i_, 0)),
            _row_spec(BLK, d),
            _row_spec(BLK, 1),
            _row_spec(BLK, d),
            _full_spec((d, d)),
        ]
        consts = (vec(bg[i]), vec(bl[i]), vec(running_mean[i]),
                  vec(running_var[i]), vec(gamma[i]), vec(beta[i]))
        if not last:
            h, ap = pl.pallas_call(
                _combine_body,
                grid=grid,
                in_specs=common_in + [_full_spec((d, d))] + vspecs,
                out_specs=[_row_spec(BLK, d), _row_spec(BLK, d)],
                out_shape=[
                    jax.ShapeDtypeStruct((n, d), jnp.float32),
                    jax.ShapeDtypeStruct((n, d), jnp.float32),
                ],
            )(agg, ap, dis, h, Wl[i], Wg[i + 1], *consts)
        else:
            y = pl.pallas_call(
                _final_body,
                grid=grid,
                in_specs=common_in + [_full_spec((d, d_out))] + vspecs
                + [_full_spec((1, d_out))],
                out_specs=_row_spec(BLK, d_out),
                out_shape=jax.ShapeDtypeStruct((n, d_out), jnp.float32),
            )(agg, ap, dis, h, Wl[i], Wp, *consts, vec(bp))
    return y


# speedup vs baseline: 1.3933x; 1.3933x over previous
import functools

import jax
import jax.numpy as jnp
from jax import lax
from jax.experimental import pallas as pl
from jax.experimental.pallas import tpu as pltpu
from jax.experimental.pallas import tpu_sc as plsc

NC = 2
NS = 16
NW = NC * NS
K = 128
BN_EPS = 1e-5
BLK = 1000


def _sc_mesh():
    return plsc.VectorSubcoreMesh(
        core_axis_name="c", subcore_axis_name="s", num_cores=NC, num_subcores=NS
    )


def _make_agg_kernel(ch_per_w, n_acc, d):
    rpt = n_acc // NS

    @functools.partial(
        pl.kernel,
        out_type=jax.ShapeDtypeStruct((NC, n_acc, d), jnp.float32),
        mesh=_sc_mesh(),
        scratch_types=[
            pltpu.VMEM((2, K), jnp.int32),
            pltpu.VMEM((K, d), jnp.float32),
            pltpu.VMEM_SHARED((n_acc, d), jnp.float32),
            pltpu.SemaphoreType.DMA,
        ],
    )
    def agg_kernel(ap_hbm, eidx_hbm, zeros_hbm, out_hbm, idx, rows, acc, sem):
        c = lax.axis_index("c")
        s = lax.axis_index("s")
        w = s * NC + c
        pltpu.sync_copy(zeros_hbm, acc.at[pl.ds(s * rpt, rpt)])
        plsc.subcore_barrier()

        def body(ch, carry):
            pltpu.sync_copy(eidx_hbm.at[w, ch], idx)
            pltpu.async_copy(ap_hbm.at[idx.at[0]], rows, sem).wait()
            pltpu.sync_copy(rows, acc.at[idx.at[1]], add=True)
            return carry

        lax.fori_loop(0, ch_per_w, body, 0)
        plsc.subcore_barrier()
        pltpu.sync_copy(
            acc.at[pl.ds(s * rpt, rpt)], out_hbm.at[c, pl.ds(s * rpt, rpt)]
        )

    return agg_kernel


def _make_deg_kernel(ch_per_w, n_acc, d):
    rpt = n_acc // NS

    @functools.partial(
        pl.kernel,
        out_type=jax.ShapeDtypeStruct((NC, n_acc, d), jnp.float32),
        mesh=_sc_mesh(),
        scratch_types=[
            pltpu.VMEM((1, K), jnp.int32),
            pltpu.VMEM((K, d), jnp.float32),
            pltpu.VMEM_SHARED((n_acc, d), jnp.float32),
        ],
    )
    def deg_kernel(eidx_hbm, ones_hbm, zeros_hbm, out_hbm, idx, ones_v, acc):
        c = lax.axis_index("c")
        s = lax.axis_index("s")
        w = s * NC + c
        pltpu.sync_copy(ones_hbm, ones_v)
        pltpu.sync_copy(zeros_hbm, acc.at[pl.ds(s * rpt, rpt)])
        plsc.subcore_barrier()

        def body(ch, carry):
            pltpu.sync_copy(eidx_hbm.at[w, ch, 1], idx.at[0])
            pltpu.sync_copy(ones_v, acc.at[idx.at[0]], add=True)
            return carry

        lax.fori_loop(0, ch_per_w, body, 0)
        plsc.subcore_barrier()
        pltpu.sync_copy(
            acc.at[pl.ds(s * rpt, rpt)], out_hbm.at[c, pl.ds(s * rpt, rpt)]
        )

    return deg_kernel


def _entry_body(dega_ref, x_ref, wg_ref, dis_ref, ap_ref):
    dw = dega_ref[...]
    deg = dw[0, :, 0:1] + dw[1, :, 0:1] + 1.0
    dis = lax.rsqrt(deg)
    dis_ref[...] = dis
    ap_ref[...] = (
        jnp.dot(x_ref[...], wg_ref[...], preferred_element_type=jnp.float32) * dis
    )


def _combine_body(agg_ref, ap_ref, dis_ref, h_ref, wl_ref, wgn_ref,
                  bg_ref, bl_ref, mean_ref, var_ref, gamma_ref, beta_ref,
                  hn_ref, apn_ref):
    ag = agg_ref[...]
    dis = dis_ref[...]
    pre = (ag[0] + ag[1] + ap_ref[...]) * dis + bg_ref[...]
    pre += jnp.dot(h_ref[...], wl_ref[...], preferred_element_type=jnp.float32)
    pre += bl_ref[...]
    scale = gamma_ref[...] * lax.rsqrt(var_ref[...] + BN_EPS)
    hn = jnp.maximum((pre - mean_ref[...]) * scale + beta_ref[...], 0.0)
    hn_ref[...] = hn
    apn_ref[...] = (
        jnp.dot(hn, wgn_ref[...], preferred_element_type=jnp.float32) * dis
    )


def _final_body(agg_ref, ap_ref, dis_ref, h_ref, wl_ref, wp_ref,
                bg_ref, bl_ref, mean_ref, var_ref, gamma_ref, beta_ref,
                bp_ref, y_ref):
    ag = agg_ref[...]
    dis = dis_ref[...]
    pre = (ag[0] + ag[1] + ap_ref[...]) * dis + bg_ref[...]
    pre += jnp.dot(h_ref[...], wl_ref[...], preferred_element_type=jnp.float32)
    pre += bl_ref[...]
    scale = gamma_ref[...] * lax.rsqrt(var_ref[...] + BN_EPS)
    hn = jnp.maximum((pre - mean_ref[...]) * scale + beta_ref[...], 0.0)
    y_ref[...] = (
        jnp.dot(hn, wp_ref[...], preferred_element_type=jnp.float32) + bp_ref[...]
    )


def _row_spec(b, d):
    return pl.BlockSpec((b, d), lambda i: (i, 0))


def _full_spec(shape):
    ndim = len(shape)
    return pl.BlockSpec(shape, lambda i: (0,) * ndim)


def kernel(x, edge_index, Wg, bg, Wl, bl, gamma, beta, running_mean,
           running_var, Wp, bp):
    n, d = x.shape
    e = edge_index.shape[1]
    d_out = Wp.shape[1]
    n_layers = Wg.shape[0]

    ch_per_w = -(-e // (NW * K))
    e_pad = NW * ch_per_w * K
    n_acc = -(-(n + 1) // (NS * 8)) * (NS * 8)
    rpt = n_acc // NS

    src = edge_index[0].astype(jnp.int32)
    dst = edge_index[1].astype(jnp.int32)
    pad = e_pad - e
    esrc = jnp.concatenate([src, jnp.zeros((pad,), jnp.int32)]).reshape(NW, ch_per_w, K)
    edst = jnp.concatenate([dst, jnp.full((pad,), n, jnp.int32)]).reshape(NW, ch_per_w, K)
    eidx = jnp.stack([esrc, edst], axis=2)

    zeros_agg = jnp.zeros((rpt, d), jnp.float32)
    ones_kd = jnp.ones((K, d), jnp.float32)

    agg_kernel = _make_agg_kernel(ch_per_w, n_acc, d)

    dega = _make_deg_kernel(ch_per_w, n_acc, d)(eidx, ones_kd, zeros_agg)

    grid = (n // BLK,)
    dis, ap = pl.pallas_call(
        _entry_body,
        grid=grid,
        in_specs=[
            pl.BlockSpec((NC, BLK, d), lambda i: (0, i, 0)),
            _row_spec(BLK, d),
            _full_spec((d, d)),
        ],
        out_specs=[_row_spec(BLK, 1), _row_spec(BLK, d)],
        out_shape=[
            jax.ShapeDtypeStruct((n, 1), jnp.float32),
            jax.ShapeDtypeStruct((n, d), jnp.float32),
        ],
    )(dega, x, Wg[0])

    vec = lambda a: a.reshape(1, -1)
    h = x
    for i in range(n_layers):
        agg = agg_kernel(ap, eidx, zeros_agg)
        last = i == n_layers - 1
        vspecs = [_full_spec((1, d))] * 6
        common_in = [
            pl.BlockSpec((NC, BLK, d), lambda i_: (0, i_, 0)),
            _row_spec(BLK, d),
            _row_spec(BLK, 1),
            _row_spec(BLK, d),
            _full_spec((d, d)),
        ]
        consts = (vec(bg[i]), vec(bl[i]), vec(running_mean[i]),
                  vec(running_var[i]), vec(gamma[i]), vec(beta[i]))
        if not last:
            h, ap = pl.pallas_call(
                _combine_body,
                grid=grid,
                in_specs=common_in + [_full_spec((d, d))] + vspecs,
                out_specs=[_row_spec(BLK, d), _row_spec(BLK, d)],
                out_shape=[
                    jax.ShapeDtypeStruct((n, d), jnp.float32),
                    jax.ShapeDtypeStruct((n, d), jnp.float32),
                ],
            )(agg, ap, dis, h, Wl[i], Wg[i + 1], *consts)
        else:
            y = pl.pallas_call(
                _final_body,
                grid=grid,
                in_specs=common_in + [_full_spec((d, d_out))] + vspecs
                + [_full_spec((1, d_out))],
                out_specs=_row_spec(BLK, d_out),
                out_shape=jax.ShapeDtypeStruct((n, d_out), jnp.float32),
            )(agg, ap, dis, h, Wl[i], Wp, *consts, vec(bp))
    return y
